# Initial kernel scaffold; baseline (speedup 1.0000x reference)
import jax
import jax.numpy as jnp
from jax.experimental import pallas as pl


def kernel(x, edge_index, batch, eps1, eps2, eps3, W1a, b1a, W1b, b1b, W2a, b2a, W2b, b2b, W3a, b3a, W3b, b3b, g1, be1, g2, be2, g3, be3, fc1W, fc1b, fc2W, fc2b):
    raise NotImplementedError("write your pallas kernel here")


# speedup vs baseline: 5.7958x; 5.7958x over previous
import functools

import jax
import jax.numpy as jnp
from jax import lax
from jax.experimental import pallas as pl
from jax.experimental.pallas import tpu as pltpu
from jax.experimental.pallas import tpu_sc as plsc

_NC = 2
_NS = 16
_NW = _NC * _NS
_CHUNK = 128


def _make_agg(npad, width, chunks):
    rows_per_tile = npad // _NS
    mesh = plsc.VectorSubcoreMesh(core_axis_name="c", subcore_axis_name="s")

    @functools.partial(
        pl.kernel,
        mesh=mesh,
        out_type=jax.ShapeDtypeStruct((_NC, npad, width), jnp.float32),
        compiler_params=pltpu.CompilerParams(use_tc_tiling_on_sc=False),
        scratch_types=[
            pltpu.VMEM((chunks, _CHUNK), jnp.int32),
            pltpu.VMEM((chunks, _CHUNK), jnp.int32),
            pltpu.VMEM((_CHUNK, width), jnp.float32),
            pltpu.VMEM_SHARED((npad, width), jnp.float32),
        ],
    )
    def agg(h_hbm, src_hbm, dst_hbm, out_hbm, src_v, dst_v, rows_v, acc_sh):
        c = lax.axis_index("c")
        s = lax.axis_index("s")
        w = c * _NS + s

        pltpu.sync_copy(src_hbm.at[w], src_v)
        pltpu.sync_copy(dst_hbm.at[w], dst_v)

        def zrow(i, carry):
            for j in range(width // 16):
                rows_v[i, pl.ds(j * 16, 16)] = jnp.zeros((16,), jnp.float32)
            return carry
        lax.fori_loop(0, _CHUNK, zrow, 0)
        n_full = rows_per_tile // _CHUNK
        for b in range(n_full):
            pltpu.sync_copy(
                rows_v,
                acc_sh.at[pl.ds(s * rows_per_tile + b * _CHUNK, _CHUNK)])
        rem = rows_per_tile - n_full * _CHUNK
        if rem:
            pltpu.sync_copy(
                rows_v.at[pl.ds(0, rem)],
                acc_sh.at[pl.ds(s * rows_per_tile + n_full * _CHUNK, rem)])
        plsc.subcore_barrier()

        def body(ci, carry):
            pltpu.sync_copy(h_hbm.at[src_v.at[ci]], rows_v)
            pltpu.sync_copy(rows_v, acc_sh.at[dst_v.at[ci]], add=True)
            return carry
        lax.fori_loop(0, chunks, body, 0)
        plsc.subcore_barrier()

        pltpu.sync_copy(
            acc_sh.at[pl.ds(s * rows_per_tile, rows_per_tile)],
            out_hbm.at[c, pl.ds(s * rows_per_tile, rows_per_tile)])

    return agg


def _gin_core(h, p_ref, eps_ref, wa_ref, ba_ref, wb_ref, bb_ref, g_ref, be_ref):
    n = h.shape[0]
    t = (1.0 + eps_ref[0, 0]) * h + (p_ref[0, :n, :] + p_ref[1, :n, :])
    u = jnp.maximum(
        jnp.dot(t, wa_ref[...], preferred_element_type=jnp.float32)
        + ba_ref[...], 0.0)
    v = jnp.dot(u, wb_ref[...], preferred_element_type=jnp.float32) + bb_ref[...]
    hh = jnp.maximum(v, 0.0)
    mu = jnp.mean(hh, axis=0, keepdims=True)
    var = jnp.mean((hh - mu) ** 2, axis=0, keepdims=True)
    return (hh - mu) * lax.rsqrt(var + 1e-5) * g_ref[...] + be_ref[...]


def _layer_body(h_ref, p_ref, eps_ref, wa_ref, ba_ref, wb_ref, bb_ref, g_ref,
                be_ref, o_ref):
    o_ref[...] = _gin_core(h_ref[...], p_ref, eps_ref, wa_ref, ba_ref, wb_ref,
                           bb_ref, g_ref, be_ref)


def _final_body(h_ref, p_ref, eps_ref, wa_ref, ba_ref, wb_ref, bb_ref, g_ref,
                be_ref, batch_ref, fc1w_ref, fc1b_ref, fc2w_ref, fc2b_ref,
                o_ref):
    hn = _gin_core(h_ref[...], p_ref, eps_ref, wa_ref, ba_ref, wb_ref, bb_ref,
                   g_ref, be_ref)
    n = hn.shape[0]
    g_segs = o_ref.shape[0]
    onehot = (batch_ref[...] == lax.broadcasted_iota(
        jnp.int32, (n, g_segs), 1)).astype(jnp.float32)
    sums = lax.dot_general(onehot, hn, (((0,), (0,)), ((), ())),
                           preferred_element_type=jnp.float32,
                           precision=lax.Precision.HIGHEST)
    cnt = jnp.sum(onehot, axis=0)
    pooled = sums / jnp.maximum(cnt, 1.0)[:, None]
    oo = jnp.maximum(
        jnp.dot(pooled, fc1w_ref[...], preferred_element_type=jnp.float32)
        + fc1b_ref[...], 0.0)
    o_ref[...] = jnp.dot(oo, fc2w_ref[...],
                         preferred_element_type=jnp.float32) + fc2b_ref[...]


def kernel(x, edge_index, batch, eps1, eps2, eps3, W1a, b1a, W1b, b1b,
           W2a, b2a, W2b, b2b, W3a, b3a, W3b, b3b, g1, be1, g2, be2,
           g3, be3, fc1W, fc1b, fc2W, fc2b):
    n, d = x.shape
    h = W1a.shape[1]
    e = edge_index.shape[1]
    npad = ((n + 16 + 127) // 128) * 128

    ept_raw = e // _NW
    chunks = -(-ept_raw // _CHUNK)
    pad = chunks * _CHUNK - ept_raw
    src = edge_index[0].reshape(_NW, ept_raw)
    dst = edge_index[1].reshape(_NW, ept_raw)
    if pad:
        pad_src = jnp.zeros((_NW, pad), jnp.int32)
        pad_dst = jnp.broadcast_to(
            (jnp.arange(pad, dtype=jnp.int32) % 16) + n, (_NW, pad))
        src = jnp.concatenate([src, pad_src], axis=1)
        dst = jnp.concatenate([dst, pad_dst], axis=1)
    src3 = src.reshape(_NW, chunks, _CHUNK)
    dst3 = dst.reshape(_NW, chunks, _CHUNK)

    agg_d = _make_agg(npad, d, chunks)
    agg_h = _make_agg(npad, h, chunks)

    layer = pl.pallas_call(
        _layer_body, out_shape=jax.ShapeDtypeStruct((n, h), jnp.float32))
    g_segs = 64
    final = pl.pallas_call(
        _final_body,
        out_shape=jax.ShapeDtypeStruct((g_segs, fc2W.shape[1]), jnp.float32))

    r2 = lambda v: v.reshape(1, -1)
    e1, e2, e3 = (jnp.reshape(v, (1, 1)) for v in (eps1, eps2, eps3))

    p1 = agg_d(x, src3, dst3)
    h1 = layer(x, p1, e1, W1a, r2(b1a), W1b, r2(b1b), r2(g1), r2(be1))
    p2 = agg_h(h1, src3, dst3)
    h2 = layer(h1, p2, e2, W2a, r2(b2a), W2b, r2(b2b), r2(g2), r2(be2))
    p3 = agg_h(h2, src3, dst3)
    out = final(h2, p3, e3, W3a, r2(b3a), W3b, r2(b3b), r2(g3), r2(be3),
                batch.reshape(n, 1), fc1W, r2(fc1b), fc2W, r2(fc2b))
    return out


# speedup vs baseline: 7.0842x; 1.2223x over previous
import functools

import jax
import jax.numpy as jnp
from jax import lax
from jax.experimental import pallas as pl
from jax.experimental.pallas import tpu as pltpu
from jax.experimental.pallas import tpu_sc as plsc

_NC = 2
_NS = 16
_NW = _NC * _NS
_CHUNK = 128
_NB = 4


def _make_agg(npad, width, chunks):
    rows_per_tile = npad // _NS
    mesh = plsc.VectorSubcoreMesh(core_axis_name="c", subcore_axis_name="s")

    @functools.partial(
        pl.kernel,
        mesh=mesh,
        out_type=jax.ShapeDtypeStruct((_NC, npad, width), jnp.float32),
        compiler_params=pltpu.CompilerParams(use_tc_tiling_on_sc=False),
        scratch_types=[
            pltpu.VMEM((chunks, _CHUNK), jnp.int32),
            pltpu.VMEM((chunks, _CHUNK), jnp.int32),
            pltpu.VMEM((_NB, _CHUNK, width), jnp.float32),
            pltpu.VMEM_SHARED((npad, width), jnp.float32),
            pltpu.SemaphoreType.DMA,
            pltpu.SemaphoreType.DMA,
            pltpu.SemaphoreType.DMA,
        ],
    )
    def agg(h_hbm, src_hbm, dst_hbm, out_hbm, src_v, dst_v, rows_v, acc_sh,
            gsem, ssem, isem):
        c = lax.axis_index("c")
        s = lax.axis_index("s")
        w = c * _NS + s

        pltpu.async_copy(src_hbm.at[w], src_v, isem)
        pltpu.async_copy(dst_hbm.at[w], dst_v, isem)

        def zrow(i, carry):
            for j in range(width // 16):
                rows_v[0, i, pl.ds(j * 16, 16)] = jnp.zeros((16,), jnp.float32)
            return carry
        lax.fori_loop(0, _CHUNK, zrow, 0)
        n_full = rows_per_tile // _CHUNK
        for b in range(n_full):
            pltpu.sync_copy(
                rows_v.at[0],
                acc_sh.at[pl.ds(s * rows_per_tile + b * _CHUNK, _CHUNK)])
        rem = rows_per_tile - n_full * _CHUNK
        if rem:
            pltpu.sync_copy(
                rows_v.at[0, pl.ds(0, rem)],
                acc_sh.at[pl.ds(s * rows_per_tile + n_full * _CHUNK, rem)])
        pltpu.make_async_copy(src_hbm.at[w], src_v, isem).wait()
        pltpu.make_async_copy(dst_hbm.at[w], dst_v, isem).wait()
        plsc.subcore_barrier()

        def gather(ci, b):
            pltpu.async_copy(h_hbm.at[src_v.at[ci]], rows_v.at[b], gsem)

        def gather_wait(ci, b):
            pltpu.make_async_copy(h_hbm.at[src_v.at[ci]], rows_v.at[b],
                                  gsem).wait()

        def scat(ci, b):
            pltpu.async_copy(rows_v.at[b], acc_sh.at[dst_v.at[ci]], ssem,
                             add=True)

        def scat_wait(ci, b):
            pltpu.make_async_copy(rows_v.at[b], acc_sh.at[dst_v.at[ci]],
                                  ssem).wait()

        gather(0, 0)
        gather(1, 1)

        def body(ci, carry):
            b = lax.rem(ci, _NB)
            gather_wait(ci, b)
            scat(ci, b)

            @pl.when(ci >= 2)
            def _():
                cp = ci - 2
                scat_wait(cp, lax.rem(cp, _NB))

            @pl.when(ci + 2 < chunks)
            def _():
                cn = ci + 2
                gather(cn, lax.rem(cn, _NB))
            return carry
        lax.fori_loop(0, chunks, body, 0)
        scat_wait(chunks - 2, (chunks - 2) % _NB)
        scat_wait(chunks - 1, (chunks - 1) % _NB)
        plsc.subcore_barrier()

        pltpu.sync_copy(
            acc_sh.at[pl.ds(s * rows_per_tile, rows_per_tile)],
            out_hbm.at[c, pl.ds(s * rows_per_tile, rows_per_tile)])

    return agg


def _gin_tail(t, wa_ref, ba_ref, wb_ref, bb_ref, g_ref, be_ref):
    u = jnp.maximum(
        jnp.dot(t, wa_ref[...], preferred_element_type=jnp.float32)
        + ba_ref[...], 0.0)
    v = jnp.dot(u, wb_ref[...], preferred_element_type=jnp.float32) + bb_ref[...]
    hh = jnp.maximum(v, 0.0)
    mu = jnp.mean(hh, axis=0, keepdims=True)
    var = jnp.mean((hh - mu) ** 2, axis=0, keepdims=True)
    return (hh - mu) * lax.rsqrt(var + 1e-5) * g_ref[...] + be_ref[...]


def _gin_core(h, p_ref, eps_ref, wa_ref, ba_ref, wb_ref, bb_ref, g_ref, be_ref):
    n = h.shape[0]
    t = (1.0 + eps_ref[0, 0]) * h + (p_ref[0, :n, :] + p_ref[1, :n, :])
    return _gin_tail(t, wa_ref, ba_ref, wb_ref, bb_ref, g_ref, be_ref)


def _layer_body(h_ref, p_ref, eps_ref, wa_ref, ba_ref, wb_ref, bb_ref, g_ref,
                be_ref, o_ref):
    o_ref[...] = _gin_core(h_ref[...], p_ref, eps_ref, wa_ref, ba_ref, wb_ref,
                           bb_ref, g_ref, be_ref)


def _layer1_body(h_ref, plo_ref, phi_ref, eps_ref, wa_ref, ba_ref, wb_ref,
                 bb_ref, g_ref, be_ref, o_ref):
    hx = h_ref[...]
    n = hx.shape[0]
    p = jnp.concatenate(
        [plo_ref[0, :n, :] + plo_ref[1, :n, :],
         phi_ref[0, :n, :] + phi_ref[1, :n, :]], axis=1)
    t = (1.0 + eps_ref[0, 0]) * hx + p
    o_ref[...] = _gin_tail(t, wa_ref, ba_ref, wb_ref, bb_ref, g_ref, be_ref)


def _final_body(h_ref, p_ref, eps_ref, wa_ref, ba_ref, wb_ref, bb_ref, g_ref,
                be_ref, batch_ref, fc1w_ref, fc1b_ref, fc2w_ref, fc2b_ref,
                o_ref):
    hn = _gin_core(h_ref[...], p_ref, eps_ref, wa_ref, ba_ref, wb_ref, bb_ref,
                   g_ref, be_ref)
    n = hn.shape[0]
    g_segs = o_ref.shape[0]
    onehot = (batch_ref[...] == lax.broadcasted_iota(
        jnp.int32, (n, g_segs), 1)).astype(jnp.float32)
    sums = lax.dot_general(onehot, hn, (((0,), (0,)), ((), ())),
                           preferred_element_type=jnp.float32,
                           precision=lax.Precision.HIGHEST)
    cnt = jnp.sum(onehot, axis=0)
    pooled = sums / jnp.maximum(cnt, 1.0)[:, None]
    oo = jnp.maximum(
        jnp.dot(pooled, fc1w_ref[...], preferred_element_type=jnp.float32)
        + fc1b_ref[...], 0.0)
    o_ref[...] = jnp.dot(oo, fc2w_ref[...],
                         preferred_element_type=jnp.float32) + fc2b_ref[...]


def kernel(x, edge_index, batch, eps1, eps2, eps3, W1a, b1a, W1b, b1b,
           W2a, b2a, W2b, b2b, W3a, b3a, W3b, b3b, g1, be1, g2, be2,
           g3, be3, fc1W, fc1b, fc2W, fc2b):
    n, d = x.shape
    h = W1a.shape[1]
    e = edge_index.shape[1]
    npad = ((n + 16 + 127) // 128) * 128

    ept_raw = e // _NW
    chunks = -(-ept_raw // _CHUNK)
    pad = chunks * _CHUNK - ept_raw
    src = edge_index[0].reshape(_NW, ept_raw)
    dst = edge_index[1].reshape(_NW, ept_raw)
    if pad:
        pad_src = jnp.zeros((_NW, pad), jnp.int32)
        pad_dst = jnp.broadcast_to(
            (jnp.arange(pad, dtype=jnp.int32) % 16) + n, (_NW, pad))
        src = jnp.concatenate([src, pad_src], axis=1)
        dst = jnp.concatenate([dst, pad_dst], axis=1)
    src3 = src.reshape(_NW, chunks, _CHUNK)
    dst3 = dst.reshape(_NW, chunks, _CHUNK)

    agg_h = _make_agg(npad, h, chunks)

    layer = pl.pallas_call(
        _layer_body, out_shape=jax.ShapeDtypeStruct((n, h), jnp.float32))
    layer1 = pl.pallas_call(
        _layer1_body, out_shape=jax.ShapeDtypeStruct((n, h), jnp.float32))
    g_segs = 64
    final = pl.pallas_call(
        _final_body,
        out_shape=jax.ShapeDtypeStruct((g_segs, fc2W.shape[1]), jnp.float32))

    r2 = lambda v: v.reshape(1, -1)
    e1, e2, e3 = (jnp.reshape(v, (1, 1)) for v in (eps1, eps2, eps3))

    p1a = agg_h(x[:, :h], src3, dst3)
    p1b = agg_h(x[:, h:], src3, dst3)
    h1 = layer1(x, p1a, p1b, e1, W1a, r2(b1a), W1b, r2(b1b), r2(g1), r2(be1))
    p2 = agg_h(h1, src3, dst3)
    h2 = layer(h1, p2, e2, W2a, r2(b2a), W2b, r2(b2b), r2(g2), r2(be2))
    p3 = agg_h(h2, src3, dst3)
    out = final(h2, p3, e3, W3a, r2(b3a), W3b, r2(b3b), r2(g3), r2(be3),
                batch.reshape(n, 1), fc1W, r2(fc1b), fc2W, r2(fc2b))
    return out


# speedup vs baseline: 7.3666x; 1.0399x over previous
import functools

import jax
import jax.numpy as jnp
from jax import lax
from jax.experimental import pallas as pl
from jax.experimental.pallas import tpu as pltpu
from jax.experimental.pallas import tpu_sc as plsc

_NC = 2
_NS = 16
_NW = _NC * _NS
_CHUNK = 128
_NB = 6
_GD = _NB // 2


def _make_agg(npad, width, chunks):
    rows_per_tile = npad // _NS
    mesh = plsc.VectorSubcoreMesh(core_axis_name="c", subcore_axis_name="s")

    @functools.partial(
        pl.kernel,
        mesh=mesh,
        out_type=jax.ShapeDtypeStruct((_NC, npad, width), jnp.float32),
        compiler_params=pltpu.CompilerParams(use_tc_tiling_on_sc=False),
        scratch_types=[
            pltpu.VMEM((chunks, _CHUNK), jnp.int32),
            pltpu.VMEM((chunks, _CHUNK), jnp.int32),
            pltpu.VMEM((_NB, _CHUNK, width), jnp.float32),
            pltpu.VMEM_SHARED((npad, width), jnp.float32),
            pltpu.SemaphoreType.DMA,
            pltpu.SemaphoreType.DMA,
            pltpu.SemaphoreType.DMA,
        ],
    )
    def agg(h_hbm, src_hbm, dst_hbm, out_hbm, src_v, dst_v, rows_v, acc_sh,
            gsem, ssem, isem):
        c = lax.axis_index("c")
        s = lax.axis_index("s")
        w = c * _NS + s

        pltpu.async_copy(src_hbm.at[w], src_v, isem)
        pltpu.async_copy(dst_hbm.at[w], dst_v, isem)

        def zrow(i, carry):
            for j in range(width // 16):
                rows_v[0, i, pl.ds(j * 16, 16)] = jnp.zeros((16,), jnp.float32)
            return carry
        lax.fori_loop(0, _CHUNK, zrow, 0)
        n_full = rows_per_tile // _CHUNK
        for b in range(n_full):
            pltpu.sync_copy(
                rows_v.at[0],
                acc_sh.at[pl.ds(s * rows_per_tile + b * _CHUNK, _CHUNK)])
        rem = rows_per_tile - n_full * _CHUNK
        if rem:
            pltpu.sync_copy(
                rows_v.at[0, pl.ds(0, rem)],
                acc_sh.at[pl.ds(s * rows_per_tile + n_full * _CHUNK, rem)])
        pltpu.make_async_copy(src_hbm.at[w], src_v, isem).wait()
        pltpu.make_async_copy(dst_hbm.at[w], dst_v, isem).wait()
        plsc.subcore_barrier()

        def gather(ci, b):
            pltpu.async_copy(h_hbm.at[src_v.at[ci]], rows_v.at[b], gsem)

        def gather_wait(ci, b):
            pltpu.make_async_copy(h_hbm.at[src_v.at[ci]], rows_v.at[b],
                                  gsem).wait()

        def scat(ci, b):
            pltpu.async_copy(rows_v.at[b], acc_sh.at[dst_v.at[ci]], ssem,
                             add=True)

        def scat_wait(ci, b):
            pltpu.make_async_copy(rows_v.at[b], acc_sh.at[dst_v.at[ci]],
                                  ssem).wait()

        for b0 in range(_GD):
            gather(b0, b0)

        def body(ci, carry):
            b = lax.rem(ci, _NB)
            gather_wait(ci, b)
            scat(ci, b)

            @pl.when(ci >= _GD)
            def _():
                cp = ci - _GD
                scat_wait(cp, lax.rem(cp, _NB))

            @pl.when(ci + _GD < chunks)
            def _():
                cn = ci + _GD
                gather(cn, lax.rem(cn, _NB))
            return carry
        lax.fori_loop(0, chunks, body, 0)
        for k in range(max(chunks - _GD, 0), chunks):
            scat_wait(k, k % _NB)
        plsc.subcore_barrier()

        pltpu.sync_copy(
            acc_sh.at[pl.ds(s * rows_per_tile, rows_per_tile)],
            out_hbm.at[c, pl.ds(s * rows_per_tile, rows_per_tile)])

    return agg


def _gin_tail(t, wa_ref, ba_ref, wb_ref, bb_ref, g_ref, be_ref):
    u = jnp.maximum(
        jnp.dot(t, wa_ref[...], preferred_element_type=jnp.float32)
        + ba_ref[...], 0.0)
    v = jnp.dot(u, wb_ref[...], preferred_element_type=jnp.float32) + bb_ref[...]
    hh = jnp.maximum(v, 0.0)
    mu = jnp.mean(hh, axis=0, keepdims=True)
    var = jnp.mean((hh - mu) ** 2, axis=0, keepdims=True)
    return (hh - mu) * lax.rsqrt(var + 1e-5) * g_ref[...] + be_ref[...]


def _gin_core(h, p_ref, eps_ref, wa_ref, ba_ref, wb_ref, bb_ref, g_ref, be_ref):
    n = h.shape[0]
    t = (1.0 + eps_ref[0, 0]) * h + (p_ref[0, :n, :] + p_ref[1, :n, :])
    return _gin_tail(t, wa_ref, ba_ref, wb_ref, bb_ref, g_ref, be_ref)


def _layer_body(h_ref, p_ref, eps_ref, wa_ref, ba_ref, wb_ref, bb_ref, g_ref,
                be_ref, o_ref):
    o_ref[...] = _gin_core(h_ref[...], p_ref, eps_ref, wa_ref, ba_ref, wb_ref,
                           bb_ref, g_ref, be_ref)


def _layer1_body(h_ref, plo_ref, phi_ref, eps_ref, wa_ref, ba_ref, wb_ref,
                 bb_ref, g_ref, be_ref, o_ref):
    hx = h_ref[...]
    n = hx.shape[0]
    p = jnp.concatenate(
        [plo_ref[0, :n, :] + plo_ref[1, :n, :],
         phi_ref[0, :n, :] + phi_ref[1, :n, :]], axis=1)
    t = (1.0 + eps_ref[0, 0]) * hx + p
    o_ref[...] = _gin_tail(t, wa_ref, ba_ref, wb_ref, bb_ref, g_ref, be_ref)


def _final_body(h_ref, p_ref, eps_ref, wa_ref, ba_ref, wb_ref, bb_ref, g_ref,
                be_ref, batch_ref, fc1w_ref, fc1b_ref, fc2w_ref, fc2b_ref,
                o_ref):
    hn = _gin_core(h_ref[...], p_ref, eps_ref, wa_ref, ba_ref, wb_ref, bb_ref,
                   g_ref, be_ref)
    n = hn.shape[0]
    g_segs = o_ref.shape[0]
    onehot = (batch_ref[...] == lax.broadcasted_iota(
        jnp.int32, (n, g_segs), 1)).astype(jnp.float32)
    sums = lax.dot_general(onehot, hn, (((0,), (0,)), ((), ())),
                           preferred_element_type=jnp.float32,
                           precision=lax.Precision.HIGHEST)
    cnt = jnp.sum(onehot, axis=0)
    pooled = sums / jnp.maximum(cnt, 1.0)[:, None]
    oo = jnp.maximum(
        jnp.dot(pooled, fc1w_ref[...], preferred_element_type=jnp.float32)
        + fc1b_ref[...], 0.0)
    o_ref[...] = jnp.dot(oo, fc2w_ref[...],
                         preferred_element_type=jnp.float32) + fc2b_ref[...]


def kernel(x, edge_index, batch, eps1, eps2, eps3, W1a, b1a, W1b, b1b,
           W2a, b2a, W2b, b2b, W3a, b3a, W3b, b3b, g1, be1, g2, be2,
           g3, be3, fc1W, fc1b, fc2W, fc2b):
    n, d = x.shape
    h = W1a.shape[1]
    e = edge_index.shape[1]
    npad = ((n + 16 + 127) // 128) * 128

    ept_raw = e // _NW
    chunks = -(-ept_raw // _CHUNK)
    pad = chunks * _CHUNK - ept_raw
    src = edge_index[0].reshape(_NW, ept_raw)
    dst = edge_index[1].reshape(_NW, ept_raw)
    if pad:
        pad_src = jnp.zeros((_NW, pad), jnp.int32)
        pad_dst = jnp.broadcast_to(
            (jnp.arange(pad, dtype=jnp.int32) % 16) + n, (_NW, pad))
        src = jnp.concatenate([src, pad_src], axis=1)
        dst = jnp.concatenate([dst, pad_dst], axis=1)
    src3 = src.reshape(_NW, chunks, _CHUNK)
    dst3 = dst.reshape(_NW, chunks, _CHUNK)

    agg_h = _make_agg(npad, h, chunks)

    layer = pl.pallas_call(
        _layer_body, out_shape=jax.ShapeDtypeStruct((n, h), jnp.float32))
    layer1 = pl.pallas_call(
        _layer1_body, out_shape=jax.ShapeDtypeStruct((n, h), jnp.float32))
    g_segs = 64
    final = pl.pallas_call(
        _final_body,
        out_shape=jax.ShapeDtypeStruct((g_segs, fc2W.shape[1]), jnp.float32))

    r2 = lambda v: v.reshape(1, -1)
    e1, e2, e3 = (jnp.reshape(v, (1, 1)) for v in (eps1, eps2, eps3))

    p1a = agg_h(x[:, :h], src3, dst3)
    p1b = agg_h(x[:, h:], src3, dst3)
    h1 = layer1(x, p1a, p1b, e1, W1a, r2(b1a), W1b, r2(b1b), r2(g1), r2(be1))
    p2 = agg_h(h1, src3, dst3)
    h2 = layer(h1, p2, e2, W2a, r2(b2a), W2b, r2(b2b), r2(g2), r2(be2))
    p3 = agg_h(h2, src3, dst3)
    out = final(h2, p3, e3, W3a, r2(b3a), W3b, r2(b3b), r2(g3), r2(be3),
                batch.reshape(n, 1), fc1W, r2(fc1b), fc2W, r2(fc2b))
    return out


# speedup vs baseline: 9.1883x; 1.2473x over previous
import functools

import jax
import jax.numpy as jnp
from jax import lax
from jax.experimental import pallas as pl
from jax.experimental.pallas import tpu as pltpu
from jax.experimental.pallas import tpu_sc as plsc

_NC = 2
_NS = 16
_NW = _NC * _NS
_CHUNK = 128
_NB = 6
_GD = _NB // 2


def _make_agg(npad, width, chunks):
    rows_per_tile = npad // _NS
    mesh = plsc.VectorSubcoreMesh(core_axis_name="c", subcore_axis_name="s")

    @functools.partial(
        pl.kernel,
        mesh=mesh,
        out_type=jax.ShapeDtypeStruct((_NC, npad, width), jnp.float32),
        compiler_params=pltpu.CompilerParams(use_tc_tiling_on_sc=False),
        scratch_types=[
            pltpu.VMEM((chunks, _CHUNK), jnp.int32),
            pltpu.VMEM((chunks, _CHUNK), jnp.int32),
            pltpu.VMEM((_NB, _CHUNK, width), jnp.float32),
            pltpu.VMEM_SHARED((npad, width), jnp.float32),
            pltpu.SemaphoreType.DMA,
            pltpu.SemaphoreType.DMA,
            pltpu.SemaphoreType.DMA,
        ],
    )
    def agg(h_hbm, src_hbm, dst_hbm, out_hbm, src_v, dst_v, rows_v, acc_sh,
            gsem, ssem, isem):
        c = lax.axis_index("c")
        s = lax.axis_index("s")
        w = c * _NS + s

        pltpu.async_copy(src_hbm.at[w], src_v, isem)
        pltpu.async_copy(dst_hbm.at[w], dst_v, isem)

        def zrow(i, carry):
            for j in range(width // 16):
                rows_v[0, i, pl.ds(j * 16, 16)] = jnp.zeros((16,), jnp.float32)
            return carry
        lax.fori_loop(0, _CHUNK, zrow, 0)
        n_full = rows_per_tile // _CHUNK
        for b in range(n_full):
            pltpu.sync_copy(
                rows_v.at[0],
                acc_sh.at[pl.ds(s * rows_per_tile + b * _CHUNK, _CHUNK)])
        rem = rows_per_tile - n_full * _CHUNK
        if rem:
            pltpu.sync_copy(
                rows_v.at[0, pl.ds(0, rem)],
                acc_sh.at[pl.ds(s * rows_per_tile + n_full * _CHUNK, rem)])
        pltpu.make_async_copy(src_hbm.at[w], src_v, isem).wait()
        pltpu.make_async_copy(dst_hbm.at[w], dst_v, isem).wait()
        plsc.subcore_barrier()

        def gather(ci, b):
            pltpu.async_copy(h_hbm.at[src_v.at[ci]], rows_v.at[b], gsem)

        def gather_wait(ci, b):
            pltpu.make_async_copy(h_hbm.at[src_v.at[ci]], rows_v.at[b],
                                  gsem).wait()

        def scat(ci, b):
            pltpu.async_copy(rows_v.at[b], acc_sh.at[dst_v.at[ci]], ssem,
                             add=True)

        def scat_wait(ci, b):
            pltpu.make_async_copy(rows_v.at[b], acc_sh.at[dst_v.at[ci]],
                                  ssem).wait()

        for b0 in range(_GD):
            gather(b0, b0)

        def body(ci, carry):
            b = lax.rem(ci, _NB)
            gather_wait(ci, b)
            scat(ci, b)

            @pl.when(ci >= _GD)
            def _():
                cp = ci - _GD
                scat_wait(cp, lax.rem(cp, _NB))

            @pl.when(ci + _GD < chunks)
            def _():
                cn = ci + _GD
                gather(cn, lax.rem(cn, _NB))
            return carry
        lax.fori_loop(0, chunks, body, 0)
        for k in range(max(chunks - _GD, 0), chunks):
            scat_wait(k, k % _NB)
        plsc.subcore_barrier()

        pltpu.sync_copy(
            acc_sh.at[pl.ds(s * rows_per_tile, rows_per_tile)],
            out_hbm.at[c, pl.ds(s * rows_per_tile, rows_per_tile)])

    return agg


def _make_agg_lohi(npad, width, chunks):
    rows_per_tile = npad // _NS
    mesh = plsc.VectorSubcoreMesh(core_axis_name="c", subcore_axis_name="s")

    @functools.partial(
        pl.kernel,
        mesh=mesh,
        out_type=jax.ShapeDtypeStruct((_NC, npad, width), jnp.float32),
        compiler_params=pltpu.CompilerParams(use_tc_tiling_on_sc=False),
        scratch_types=[
            pltpu.VMEM((chunks, _CHUNK), jnp.int32),
            pltpu.VMEM((chunks, _CHUNK), jnp.int32),
            pltpu.VMEM((_NB, _CHUNK, width), jnp.float32),
            pltpu.VMEM_SHARED((npad, width), jnp.float32),
            pltpu.SemaphoreType.DMA,
            pltpu.SemaphoreType.DMA,
            pltpu.SemaphoreType.DMA,
        ],
    )
    def agg(hlo_hbm, hhi_hbm, src_hbm, dst_hbm, out_hbm, src_v, dst_v, rows_v,
            acc_sh, gsem, ssem, isem):
        c = lax.axis_index("c")
        s = lax.axis_index("s")

        pltpu.async_copy(src_hbm.at[s], src_v, isem)
        pltpu.async_copy(dst_hbm.at[s], dst_v, isem)

        def zrow(i, carry):
            for j in range(width // 16):
                rows_v[0, i, pl.ds(j * 16, 16)] = jnp.zeros((16,), jnp.float32)
            return carry
        lax.fori_loop(0, _CHUNK, zrow, 0)
        n_full = rows_per_tile // _CHUNK
        for b in range(n_full):
            pltpu.sync_copy(
                rows_v.at[0],
                acc_sh.at[pl.ds(s * rows_per_tile + b * _CHUNK, _CHUNK)])
        rem = rows_per_tile - n_full * _CHUNK
        if rem:
            pltpu.sync_copy(
                rows_v.at[0, pl.ds(0, rem)],
                acc_sh.at[pl.ds(s * rows_per_tile + n_full * _CHUNK, rem)])
        pltpu.make_async_copy(src_hbm.at[s], src_v, isem).wait()
        pltpu.make_async_copy(dst_hbm.at[s], dst_v, isem).wait()
        plsc.subcore_barrier()

        def run_ring(h_hbm):
            def gather(ci, b):
                pltpu.async_copy(h_hbm.at[src_v.at[ci]], rows_v.at[b], gsem)

            def gather_wait(ci, b):
                pltpu.make_async_copy(h_hbm.at[src_v.at[ci]], rows_v.at[b],
                                      gsem).wait()

            def scat(ci, b):
                pltpu.async_copy(rows_v.at[b], acc_sh.at[dst_v.at[ci]], ssem,
                                 add=True)

            def scat_wait(ci, b):
                pltpu.make_async_copy(rows_v.at[b], acc_sh.at[dst_v.at[ci]],
                                      ssem).wait()

            for b0 in range(_GD):
                gather(b0, b0)

            def body(ci, carry):
                b = lax.rem(ci, _NB)
                gather_wait(ci, b)
                scat(ci, b)

                @pl.when(ci >= _GD)
                def _():
                    cp = ci - _GD
                    scat_wait(cp, lax.rem(cp, _NB))

                @pl.when(ci + _GD < chunks)
                def _():
                    cn = ci + _GD
                    gather(cn, lax.rem(cn, _NB))
                return carry
            lax.fori_loop(0, chunks, body, 0)
            for k in range(max(chunks - _GD, 0), chunks):
                scat_wait(k, k % _NB)

        @pl.when(c == 0)
        def _():
            run_ring(hlo_hbm)

        @pl.when(c == 1)
        def _():
            run_ring(hhi_hbm)
        plsc.subcore_barrier()

        pltpu.sync_copy(
            acc_sh.at[pl.ds(s * rows_per_tile, rows_per_tile)],
            out_hbm.at[c, pl.ds(s * rows_per_tile, rows_per_tile)])

    return agg


def _gin_tail(t, wa_ref, ba_ref, wb_ref, bb_ref, g_ref, be_ref):
    u = jnp.maximum(
        jnp.dot(t, wa_ref[...], preferred_element_type=jnp.float32)
        + ba_ref[...], 0.0)
    v = jnp.dot(u, wb_ref[...], preferred_element_type=jnp.float32) + bb_ref[...]
    hh = jnp.maximum(v, 0.0)
    mu = jnp.mean(hh, axis=0, keepdims=True)
    var = jnp.mean((hh - mu) ** 2, axis=0, keepdims=True)
    return (hh - mu) * lax.rsqrt(var + 1e-5) * g_ref[...] + be_ref[...]


def _gin_core(h, p_ref, eps_ref, wa_ref, ba_ref, wb_ref, bb_ref, g_ref, be_ref):
    n = h.shape[0]
    t = (1.0 + eps_ref[0, 0]) * h + (p_ref[0, :n, :] + p_ref[1, :n, :])
    return _gin_tail(t, wa_ref, ba_ref, wb_ref, bb_ref, g_ref, be_ref)


def _layer_body(h_ref, p_ref, eps_ref, wa_ref, ba_ref, wb_ref, bb_ref, g_ref,
                be_ref, o_ref):
    o_ref[...] = _gin_core(h_ref[...], p_ref, eps_ref, wa_ref, ba_ref, wb_ref,
                           bb_ref, g_ref, be_ref)


def _layer1_body(h_ref, p_ref, eps_ref, wa_ref, ba_ref, wb_ref,
                 bb_ref, g_ref, be_ref, o_ref):
    hx = h_ref[...]
    n = hx.shape[0]
    p = jnp.concatenate(
        [p_ref[0, :n, :], p_ref[1, :n, :]], axis=1)
    t = (1.0 + eps_ref[0, 0]) * hx + p
    o_ref[...] = _gin_tail(t, wa_ref, ba_ref, wb_ref, bb_ref, g_ref, be_ref)


def _final_body(h_ref, p_ref, eps_ref, wa_ref, ba_ref, wb_ref, bb_ref, g_ref,
                be_ref, batch_ref, fc1w_ref, fc1b_ref, fc2w_ref, fc2b_ref,
                o_ref):
    hn = _gin_core(h_ref[...], p_ref, eps_ref, wa_ref, ba_ref, wb_ref, bb_ref,
                   g_ref, be_ref)
    n = hn.shape[0]
    g_segs = o_ref.shape[0]
    onehot = (batch_ref[...] == lax.broadcasted_iota(
        jnp.int32, (n, g_segs), 1)).astype(jnp.float32)
    sums = lax.dot_general(onehot, hn, (((0,), (0,)), ((), ())),
                           preferred_element_type=jnp.float32,
                           precision=lax.Precision.HIGHEST)
    cnt = jnp.sum(onehot, axis=0)
    pooled = sums / jnp.maximum(cnt, 1.0)[:, None]
    oo = jnp.maximum(
        jnp.dot(pooled, fc1w_ref[...], preferred_element_type=jnp.float32)
        + fc1b_ref[...], 0.0)
    o_ref[...] = jnp.dot(oo, fc2w_ref[...],
                         preferred_element_type=jnp.float32) + fc2b_ref[...]


def kernel(x, edge_index, batch, eps1, eps2, eps3, W1a, b1a, W1b, b1b,
           W2a, b2a, W2b, b2b, W3a, b3a, W3b, b3b, g1, be1, g2, be2,
           g3, be3, fc1W, fc1b, fc2W, fc2b):
    n, d = x.shape
    h = W1a.shape[1]
    e = edge_index.shape[1]
    npad = ((n + 16 + 127) // 128) * 128

    ept_raw = e // _NW
    chunks = -(-ept_raw // _CHUNK)
    pad = chunks * _CHUNK - ept_raw
    src = edge_index[0].reshape(_NW, ept_raw)
    dst = edge_index[1].reshape(_NW, ept_raw)
    if pad:
        pad_src = jnp.zeros((_NW, pad), jnp.int32)
        pad_dst = jnp.broadcast_to(
            (jnp.arange(pad, dtype=jnp.int32) % 16) + n, (_NW, pad))
        src = jnp.concatenate([src, pad_src], axis=1)
        dst = jnp.concatenate([dst, pad_dst], axis=1)
    src3 = src.reshape(_NW, chunks, _CHUNK)
    dst3 = dst.reshape(_NW, chunks, _CHUNK)

    agg_h = _make_agg(npad, h, chunks)

    ept1_raw = e // _NS
    chunks1 = -(-ept1_raw // _CHUNK)
    pad1 = chunks1 * _CHUNK - ept1_raw
    src1 = edge_index[0].reshape(_NS, ept1_raw)
    dst1 = edge_index[1].reshape(_NS, ept1_raw)
    if pad1:
        pad_src1 = jnp.zeros((_NS, pad1), jnp.int32)
        pad_dst1 = jnp.broadcast_to(
            (jnp.arange(pad1, dtype=jnp.int32) % 16) + n, (_NS, pad1))
        src1 = jnp.concatenate([src1, pad_src1], axis=1)
        dst1 = jnp.concatenate([dst1, pad_dst1], axis=1)
    src31 = src1.reshape(_NS, chunks1, _CHUNK)
    dst31 = dst1.reshape(_NS, chunks1, _CHUNK)
    agg1 = _make_agg_lohi(npad, h, chunks1)

    layer = pl.pallas_call(
        _layer_body, out_shape=jax.ShapeDtypeStruct((n, h), jnp.float32))
    layer1 = pl.pallas_call(
        _layer1_body, out_shape=jax.ShapeDtypeStruct((n, h), jnp.float32))
    g_segs = 64
    final = pl.pallas_call(
        _final_body,
        out_shape=jax.ShapeDtypeStruct((g_segs, fc2W.shape[1]), jnp.float32))

    r2 = lambda v: v.reshape(1, -1)
    e1, e2, e3 = (jnp.reshape(v, (1, 1)) for v in (eps1, eps2, eps3))

    p1 = agg1(x[:, :h], x[:, h:], src31, dst31)
    h1 = layer1(x, p1, e1, W1a, r2(b1a), W1b, r2(b1b), r2(g1), r2(be1))
    p2 = agg_h(h1, src3, dst3)
    h2 = layer(h1, p2, e2, W2a, r2(b2a), W2b, r2(b2b), r2(g2), r2(be2))
    p3 = agg_h(h2, src3, dst3)
    out = final(h2, p3, e3, W3a, r2(b3a), W3b, r2(b3b), r2(g3), r2(be3),
                batch.reshape(n, 1), fc1W, r2(fc1b), fc2W, r2(fc2b))
    return out


# speedup vs baseline: 9.2353x; 1.0051x over previous
import functools

import jax
import jax.numpy as jnp
from jax import lax
from jax.experimental import pallas as pl
from jax.experimental.pallas import tpu as pltpu
from jax.experimental.pallas import tpu_sc as plsc

_NC = 2
_NS = 16
_NW = _NC * _NS
_CHUNK = 128
_NB = 6
_GDG = 4
_GDS = _NB - _GDG


def _make_agg(npad, width, chunks):
    rows_per_tile = npad // _NS
    mesh = plsc.VectorSubcoreMesh(core_axis_name="c", subcore_axis_name="s")

    @functools.partial(
        pl.kernel,
        mesh=mesh,
        out_type=jax.ShapeDtypeStruct((_NC, npad, width), jnp.float32),
        compiler_params=pltpu.CompilerParams(use_tc_tiling_on_sc=False),
        scratch_types=[
            pltpu.VMEM((chunks, _CHUNK), jnp.int32),
            pltpu.VMEM((chunks, _CHUNK), jnp.int32),
            pltpu.VMEM((_NB, _CHUNK, width), jnp.float32),
            pltpu.VMEM_SHARED((npad, width), jnp.float32),
            pltpu.SemaphoreType.DMA,
            pltpu.SemaphoreType.DMA,
            pltpu.SemaphoreType.DMA,
        ],
    )
    def agg(h_hbm, src_hbm, dst_hbm, out_hbm, src_v, dst_v, rows_v, acc_sh,
            gsem, ssem, isem):
        c = lax.axis_index("c")
        s = lax.axis_index("s")
        w = c * _NS + s

        pltpu.async_copy(src_hbm.at[w], src_v, isem)
        pltpu.async_copy(dst_hbm.at[w], dst_v, isem)

        def zrow(i, carry):
            for j in range(width // 16):
                rows_v[0, i, pl.ds(j * 16, 16)] = jnp.zeros((16,), jnp.float32)
            return carry
        lax.fori_loop(0, _CHUNK, zrow, 0)
        n_full = rows_per_tile // _CHUNK
        for b in range(n_full):
            pltpu.sync_copy(
                rows_v.at[0],
                acc_sh.at[pl.ds(s * rows_per_tile + b * _CHUNK, _CHUNK)])
        rem = rows_per_tile - n_full * _CHUNK
        if rem:
            pltpu.sync_copy(
                rows_v.at[0, pl.ds(0, rem)],
                acc_sh.at[pl.ds(s * rows_per_tile + n_full * _CHUNK, rem)])
        pltpu.make_async_copy(src_hbm.at[w], src_v, isem).wait()
        pltpu.make_async_copy(dst_hbm.at[w], dst_v, isem).wait()
        plsc.subcore_barrier()

        def gather(ci, b):
            pltpu.async_copy(h_hbm.at[src_v.at[ci]], rows_v.at[b], gsem)

        def gather_wait(ci, b):
            pltpu.make_async_copy(h_hbm.at[src_v.at[ci]], rows_v.at[b],
                                  gsem).wait()

        def scat(ci, b):
            pltpu.async_copy(rows_v.at[b], acc_sh.at[dst_v.at[ci]], ssem,
                             add=True)

        def scat_wait(ci, b):
            pltpu.make_async_copy(rows_v.at[b], acc_sh.at[dst_v.at[ci]],
                                  ssem).wait()

        for b0 in range(_GDG):
            gather(b0, b0)

        def body(ci, carry):
            b = lax.rem(ci, _NB)
            gather_wait(ci, b)
            scat(ci, b)

            @pl.when(ci >= _GDS)
            def _():
                cp = ci - _GDS
                scat_wait(cp, lax.rem(cp, _NB))

            @pl.when(ci + _GDG < chunks)
            def _():
                cn = ci + _GDG
                gather(cn, lax.rem(cn, _NB))
            return carry
        lax.fori_loop(0, chunks, body, 0)
        for k in range(max(chunks - _GDS, 0), chunks):
            scat_wait(k, k % _NB)
        plsc.subcore_barrier()

        pltpu.sync_copy(
            acc_sh.at[pl.ds(s * rows_per_tile, rows_per_tile)],
            out_hbm.at[c, pl.ds(s * rows_per_tile, rows_per_tile)])

    return agg


def _make_agg_lohi(npad, width, chunks):
    rows_per_tile = npad // _NS
    mesh = plsc.VectorSubcoreMesh(core_axis_name="c", subcore_axis_name="s")

    @functools.partial(
        pl.kernel,
        mesh=mesh,
        out_type=jax.ShapeDtypeStruct((_NC, npad, width), jnp.float32),
        compiler_params=pltpu.CompilerParams(use_tc_tiling_on_sc=False),
        scratch_types=[
            pltpu.VMEM((chunks, _CHUNK), jnp.int32),
            pltpu.VMEM((chunks, _CHUNK), jnp.int32),
            pltpu.VMEM((_NB, _CHUNK, width), jnp.float32),
            pltpu.VMEM_SHARED((npad, width), jnp.float32),
            pltpu.SemaphoreType.DMA,
            pltpu.SemaphoreType.DMA,
            pltpu.SemaphoreType.DMA,
        ],
    )
    def agg(hlo_hbm, hhi_hbm, src_hbm, dst_hbm, out_hbm, src_v, dst_v, rows_v,
            acc_sh, gsem, ssem, isem):
        c = lax.axis_index("c")
        s = lax.axis_index("s")

        pltpu.async_copy(src_hbm.at[s], src_v, isem)
        pltpu.async_copy(dst_hbm.at[s], dst_v, isem)

        def zrow(i, carry):
            for j in range(width // 16):
                rows_v[0, i, pl.ds(j * 16, 16)] = jnp.zeros((16,), jnp.float32)
            return carry
        lax.fori_loop(0, _CHUNK, zrow, 0)
        n_full = rows_per_tile // _CHUNK
        for b in range(n_full):
            pltpu.sync_copy(
                rows_v.at[0],
                acc_sh.at[pl.ds(s * rows_per_tile + b * _CHUNK, _CHUNK)])
        rem = rows_per_tile - n_full * _CHUNK
        if rem:
            pltpu.sync_copy(
                rows_v.at[0, pl.ds(0, rem)],
                acc_sh.at[pl.ds(s * rows_per_tile + n_full * _CHUNK, rem)])
        pltpu.make_async_copy(src_hbm.at[s], src_v, isem).wait()
        pltpu.make_async_copy(dst_hbm.at[s], dst_v, isem).wait()
        plsc.subcore_barrier()

        def run_ring(h_hbm):
            def gather(ci, b):
                pltpu.async_copy(h_hbm.at[src_v.at[ci]], rows_v.at[b], gsem)

            def gather_wait(ci, b):
                pltpu.make_async_copy(h_hbm.at[src_v.at[ci]], rows_v.at[b],
                                      gsem).wait()

            def scat(ci, b):
                pltpu.async_copy(rows_v.at[b], acc_sh.at[dst_v.at[ci]], ssem,
                                 add=True)

            def scat_wait(ci, b):
                pltpu.make_async_copy(rows_v.at[b], acc_sh.at[dst_v.at[ci]],
                                      ssem).wait()

            for b0 in range(_GDG):
                gather(b0, b0)

            def body(ci, carry):
                b = lax.rem(ci, _NB)
                gather_wait(ci, b)
                scat(ci, b)

                @pl.when(ci >= _GDS)
                def _():
                    cp = ci - _GDS
                    scat_wait(cp, lax.rem(cp, _NB))

                @pl.when(ci + _GDG < chunks)
                def _():
                    cn = ci + _GDG
                    gather(cn, lax.rem(cn, _NB))
                return carry
            lax.fori_loop(0, chunks, body, 0)
            for k in range(max(chunks - _GDS, 0), chunks):
                scat_wait(k, k % _NB)

        @pl.when(c == 0)
        def _():
            run_ring(hlo_hbm)

        @pl.when(c == 1)
        def _():
            run_ring(hhi_hbm)
        plsc.subcore_barrier()

        pltpu.sync_copy(
            acc_sh.at[pl.ds(s * rows_per_tile, rows_per_tile)],
            out_hbm.at[c, pl.ds(s * rows_per_tile, rows_per_tile)])

    return agg


def _gin_tail(t, wa_ref, ba_ref, wb_ref, bb_ref, g_ref, be_ref):
    u = jnp.maximum(
        jnp.dot(t, wa_ref[...], preferred_element_type=jnp.float32)
        + ba_ref[...], 0.0)
    v = jnp.dot(u, wb_ref[...], preferred_element_type=jnp.float32) + bb_ref[...]
    hh = jnp.maximum(v, 0.0)
    mu = jnp.mean(hh, axis=0, keepdims=True)
    var = jnp.mean((hh - mu) ** 2, axis=0, keepdims=True)
    return (hh - mu) * lax.rsqrt(var + 1e-5) * g_ref[...] + be_ref[...]


def _gin_core(h, p_ref, eps_ref, wa_ref, ba_ref, wb_ref, bb_ref, g_ref, be_ref):
    n = h.shape[0]
    t = (1.0 + eps_ref[0, 0]) * h + (p_ref[0, :n, :] + p_ref[1, :n, :])
    return _gin_tail(t, wa_ref, ba_ref, wb_ref, bb_ref, g_ref, be_ref)


def _layer_body(h_ref, p_ref, eps_ref, wa_ref, ba_ref, wb_ref, bb_ref, g_ref,
                be_ref, o_ref):
    o_ref[...] = _gin_core(h_ref[...], p_ref, eps_ref, wa_ref, ba_ref, wb_ref,
                           bb_ref, g_ref, be_ref)


def _layer1_body(h_ref, p_ref, eps_ref, wa_ref, ba_ref, wb_ref,
                 bb_ref, g_ref, be_ref, o_ref):
    hx = h_ref[...]
    n = hx.shape[0]
    p = jnp.concatenate(
        [p_ref[0, :n, :], p_ref[1, :n, :]], axis=1)
    t = (1.0 + eps_ref[0, 0]) * hx + p
    o_ref[...] = _gin_tail(t, wa_ref, ba_ref, wb_ref, bb_ref, g_ref, be_ref)


def _final_body(h_ref, p_ref, eps_ref, wa_ref, ba_ref, wb_ref, bb_ref, g_ref,
                be_ref, batch_ref, fc1w_ref, fc1b_ref, fc2w_ref, fc2b_ref,
                o_ref):
    hn = _gin_core(h_ref[...], p_ref, eps_ref, wa_ref, ba_ref, wb_ref, bb_ref,
                   g_ref, be_ref)
    n = hn.shape[0]
    g_segs = o_ref.shape[0]
    onehot = (batch_ref[...] == lax.broadcasted_iota(
        jnp.int32, (n, g_segs), 1)).astype(jnp.float32)
    sums = lax.dot_general(onehot, hn, (((0,), (0,)), ((), ())),
                           preferred_element_type=jnp.float32,
                           precision=lax.Precision.HIGHEST)
    cnt = jnp.sum(onehot, axis=0)
    pooled = sums / jnp.maximum(cnt, 1.0)[:, None]
    oo = jnp.maximum(
        jnp.dot(pooled, fc1w_ref[...], preferred_element_type=jnp.float32)
        + fc1b_ref[...], 0.0)
    o_ref[...] = jnp.dot(oo, fc2w_ref[...],
                         preferred_element_type=jnp.float32) + fc2b_ref[...]


def kernel(x, edge_index, batch, eps1, eps2, eps3, W1a, b1a, W1b, b1b,
           W2a, b2a, W2b, b2b, W3a, b3a, W3b, b3b, g1, be1, g2, be2,
           g3, be3, fc1W, fc1b, fc2W, fc2b):
    n, d = x.shape
    h = W1a.shape[1]
    e = edge_index.shape[1]
    npad = ((n + 16 + 127) // 128) * 128

    ept_raw = e // _NW
    chunks = -(-ept_raw // _CHUNK)
    pad = chunks * _CHUNK - ept_raw
    src = edge_index[0].reshape(_NW, ept_raw)
    dst = edge_index[1].reshape(_NW, ept_raw)
    if pad:
        pad_src = jnp.zeros((_NW, pad), jnp.int32)
        pad_dst = jnp.broadcast_to(
            (jnp.arange(pad, dtype=jnp.int32) % 16) + n, (_NW, pad))
        src = jnp.concatenate([src, pad_src], axis=1)
        dst = jnp.concatenate([dst, pad_dst], axis=1)
    src3 = src.reshape(_NW, chunks, _CHUNK)
    dst3 = dst.reshape(_NW, chunks, _CHUNK)

    agg_h = _make_agg(npad, h, chunks)

    ept1_raw = e // _NS
    chunks1 = -(-ept1_raw // _CHUNK)
    pad1 = chunks1 * _CHUNK - ept1_raw
    src1 = edge_index[0].reshape(_NS, ept1_raw)
    dst1 = edge_index[1].reshape(_NS, ept1_raw)
    if pad1:
        pad_src1 = jnp.zeros((_NS, pad1), jnp.int32)
        pad_dst1 = jnp.broadcast_to(
            (jnp.arange(pad1, dtype=jnp.int32) % 16) + n, (_NS, pad1))
        src1 = jnp.concatenate([src1, pad_src1], axis=1)
        dst1 = jnp.concatenate([dst1, pad_dst1], axis=1)
    src31 = src1.reshape(_NS, chunks1, _CHUNK)
    dst31 = dst1.reshape(_NS, chunks1, _CHUNK)
    agg1 = _make_agg_lohi(npad, h, chunks1)

    layer = pl.pallas_call(
        _layer_body, out_shape=jax.ShapeDtypeStruct((n, h), jnp.float32))
    layer1 = pl.pallas_call(
        _layer1_body, out_shape=jax.ShapeDtypeStruct((n, h), jnp.float32))
    g_segs = 64
    final = pl.pallas_call(
        _final_body,
        out_shape=jax.ShapeDtypeStruct((g_segs, fc2W.shape[1]), jnp.float32))

    r2 = lambda v: v.reshape(1, -1)
    e1, e2, e3 = (jnp.reshape(v, (1, 1)) for v in (eps1, eps2, eps3))

    p1 = agg1(x[:, :h], x[:, h:], src31, dst31)
    h1 = layer1(x, p1, e1, W1a, r2(b1a), W1b, r2(b1b), r2(g1), r2(be1))
    p2 = agg_h(h1, src3, dst3)
    h2 = layer(h1, p2, e2, W2a, r2(b2a), W2b, r2(b2b), r2(g2), r2(be2))
    p3 = agg_h(h2, src3, dst3)
    out = final(h2, p3, e3, W3a, r2(b3a), W3b, r2(b3b), r2(g3), r2(be3),
                batch.reshape(n, 1), fc1W, r2(fc1b), fc2W, r2(fc2b))
    return out
